# baseline (speedup 1.0000x reference)
import functools

import jax
import jax.numpy as jnp
from jax import lax
from jax.experimental import pallas as pl
from jax.experimental.pallas import tpu as pltpu


def _sig(x):
    return 1.0 / (1.0 + jnp.exp(-x))


def _gates(g, H):
    i_g = _sig(g[:, 0:H])
    f_g = _sig(g[:, H:2 * H])
    g_g = jnp.tanh(g[:, 2 * H:3 * H])
    o_g = _sig(g[:, 3 * H:4 * H])
    return i_g, f_g, g_g, o_g


def _encoder_kernel(ids_ref, emb_ref, wih0_ref, whh0_ref, b0_ref,
                    w1_ref, b1_ref,
                    out_ref, hn0_ref, cn0_ref, hn1_ref, cn1_ref,
                    xg_ref, lhs1_ref,
                    *, seq_len, batch, hidden, vocab):
    S, B, H, V = seq_len, batch, hidden, vocab
    bf16 = jnp.bfloat16

    emb = emb_ref[...]
    wih0 = wih0_ref[...]
    b0 = b0_ref[...]
    rows = S * B
    chunk = 128 if rows % 128 == 0 else rows
    for mc in range(rows // chunk):
        ids_c = ids_ref[mc * chunk:(mc + 1) * chunk, :]
        iota = lax.broadcasted_iota(jnp.int32, (chunk, V), 1)
        oh = (ids_c == iota).astype(bf16)
        er = jnp.dot(oh, emb, preferred_element_type=jnp.float32)
        xg = jnp.dot(er.astype(bf16), wih0,
                     preferred_element_type=jnp.float32) + b0
        xg_ref[mc * chunk:(mc + 1) * chunk, :] = xg

    g0 = xg_ref[0:B, :]
    i0, f0, gg0, o0 = _gates(g0, H)
    c0n = i0 * gg0
    h0n = o0 * jnp.tanh(c0n)
    cn0_ref[...] = c0n
    hn0_ref[...] = h0n
    lhs1_ref[...] = jnp.concatenate(
        [h0n.astype(bf16), jnp.zeros((B, H), bf16)], axis=1)

    w1 = w1_ref[...]
    whh0 = whh0_ref[...]
    b1 = b1_ref[...]

    def body(i, carry):
        a1 = lhs1_ref[...]
        g1 = jnp.dot(a1, w1, preferred_element_type=jnp.float32) + b1
        g0 = jnp.dot(a1[:, 0:H], whh0,
                     preferred_element_type=jnp.float32) + xg_ref[pl.ds(i * B, B), :]

        i0, f0, gg0, o0 = _gates(g0, H)
        c0n = f0 * cn0_ref[...] + i0 * gg0
        h0n = o0 * jnp.tanh(c0n)
        cn0_ref[...] = c0n
        hn0_ref[...] = h0n

        i1, f1, gg1, o1 = _gates(g1, H)
        c1n = f1 * cn1_ref[...] + i1 * gg1
        h1n = o1 * jnp.tanh(c1n)
        cn1_ref[...] = c1n
        out_ref[pl.ds((i - 1) * B, B), :] = h1n

        lhs1_ref[...] = jnp.concatenate(
            [h0n.astype(bf16), h1n.astype(bf16)], axis=1)
        return carry

    cn1_ref[...] = jnp.zeros((B, H), jnp.float32)
    lax.fori_loop(1, S, body, 0)

    a1 = lhs1_ref[...]
    g1 = jnp.dot(a1, w1, preferred_element_type=jnp.float32) + b1
    i1, f1, gg1, o1 = _gates(g1, H)
    c1n = f1 * cn1_ref[...] + i1 * gg1
    h1n = o1 * jnp.tanh(c1n)
    cn1_ref[...] = c1n
    hn1_ref[...] = h1n
    out_ref[pl.ds((S - 1) * B, B), :] = h1n


def kernel(ids, embedding, w_ih_0, w_hh_0, b_ih_0, b_hh_0,
           w_ih_1, w_hh_1, b_ih_1, b_hh_1):
    B, S = ids.shape
    V, E = embedding.shape
    H = w_hh_0.shape[1]
    bf16 = jnp.bfloat16

    ids_col = jnp.transpose(ids).reshape(S * B, 1).astype(jnp.int32)
    emb_bf = embedding.astype(bf16)
    wih0_t = jnp.transpose(w_ih_0).astype(bf16)
    whh0_t = jnp.transpose(w_hh_0).astype(bf16)
    w1_t = jnp.concatenate(
        [jnp.transpose(w_ih_1), jnp.transpose(w_hh_1)], axis=0).astype(bf16)
    b0 = (b_ih_0 + b_hh_0).reshape(1, 4 * H)
    b1 = (b_ih_1 + b_hh_1).reshape(1, 4 * H)

    out_shapes = (
        jax.ShapeDtypeStruct((S * B, H), jnp.float32),
        jax.ShapeDtypeStruct((B, H), jnp.float32),
        jax.ShapeDtypeStruct((B, H), jnp.float32),
        jax.ShapeDtypeStruct((B, H), jnp.float32),
        jax.ShapeDtypeStruct((B, H), jnp.float32),
    )

    def full(x):
        n = len(x.shape)
        return pl.BlockSpec(x.shape, lambda: (0,) * n)

    inputs = (ids_col, emb_bf, wih0_t, whh0_t, b0, w1_t, b1)
    out_flat, hn0, cn0, hn1, cn1 = pl.pallas_call(
        functools.partial(_encoder_kernel, seq_len=S, batch=B, hidden=H,
                          vocab=V),
        out_shape=out_shapes,
        in_specs=[full(x) for x in inputs],
        out_specs=tuple(pl.BlockSpec(s.shape, lambda: (0,) * len(s.shape))
                        for s in out_shapes),
        scratch_shapes=[
            pltpu.VMEM((S * B, 4 * H), jnp.float32),
            pltpu.VMEM((B, 2 * H), bf16),
        ],
        compiler_params=pltpu.CompilerParams(
            dimension_semantics=()),
    )(*inputs)

    out = jnp.transpose(out_flat.reshape(S, B, H), (1, 0, 2))
    h_n = jnp.stack([hn0, hn1], axis=0)
    c_n = jnp.stack([cn0, cn1], axis=0)
    return out, (h_n, c_n)
